# baseline (speedup 1.0000x reference)
import functools

import jax
import jax.numpy as jnp
from jax import lax
from jax.experimental import pallas as pl
from jax.experimental.pallas import tpu as pltpu
from jax.experimental.pallas import tpu_sc as plsc

N_CTX = 8192
D_MODEL = 2048
BATCH = 4
NUM_WORKERS = 32
ROWS_PER_WORKER = N_CTX // NUM_WORKERS
CHUNK = 32


@functools.partial(
    pl.kernel,
    mesh=plsc.VectorSubcoreMesh(core_axis_name="c", subcore_axis_name="s"),
    out_type=jax.ShapeDtypeStruct((BATCH, N_CTX, D_MODEL), jnp.float32),
    scratch_types=[
        pltpu.VMEM((CHUNK, D_MODEL), jnp.float32),
        pltpu.SemaphoreType.DMA,
    ],
)
def _pos_broadcast(w_hbm, out_hbm, buf, sem):
    wid = lax.axis_index("s") * 2 + lax.axis_index("c")
    base = wid * ROWS_PER_WORKER

    def body(i, carry):
        r0 = base + i * CHUNK
        pltpu.sync_copy(w_hbm.at[pl.ds(r0, CHUNK)], buf)
        copies = [
            pltpu.async_copy(buf, out_hbm.at[b, pl.ds(r0, CHUNK)], sem)
            for b in range(BATCH)
        ]
        for cp in copies:
            cp.wait()
        return carry

    lax.fori_loop(0, ROWS_PER_WORKER // CHUNK, body, 0)


def kernel(tokens, W_pos):
    del tokens
    return _pos_broadcast(W_pos)
